# Initial kernel scaffold; baseline (speedup 1.0000x reference)
import jax
import jax.numpy as jnp
from jax.experimental import pallas as pl


def kernel(x, edge_index, batch, W1, b1, W2, b2, Wl, bl):
    raise NotImplementedError("write your pallas kernel here")


# speedup vs baseline: 11.9222x; 11.9222x over previous
import functools

import jax
import jax.numpy as jnp
from jax import lax
from jax.experimental import pallas as pl
from jax.experimental.pallas import tpu as pltpu
from jax.experimental.pallas import tpu_sc as plsc

N = 10000
E = 320000
D = 128
B = 128

NC = 2
NS = 16
NW = NC * NS
NPAD = 10240
ROWS_PS = NPAD // NS
EPW = E // NW
C = 80
NITER = EPW // C
BPAD = 136
NPPW = NPAD // NW
PC = 64
PITER = NPPW // PC

_sc_mesh = plsc.VectorSubcoreMesh(
    core_axis_name="c", subcore_axis_name="s", num_cores=NC, num_subcores=NS)


def _fill_rows(ref, nrows, value):
    def row(i, _):
        def col(j, _):
            ref[i, pl.ds(j * 16, 16)] = jnp.full((16,), value, jnp.float32)
            return 0
        lax.fori_loop(0, D // 16, col, 0)
        return 0
    lax.fori_loop(0, nrows, row, 0)


@functools.partial(
    pl.kernel,
    out_type=jax.ShapeDtypeStruct((NC, NS, ROWS_PS, D), jnp.float32),
    mesh=_sc_mesh,
    scratch_types=[
        pltpu.VMEM((C,), jnp.int32),
        pltpu.VMEM((C, D), jnp.float32),
        pltpu.VMEM((64, D), jnp.float32),
        pltpu.VMEM_SHARED((NPAD, D), jnp.float32),
    ],
)
def _sc_degree(dst_hbm, out_hbm, dstv, ones_v, zv, acc):
    cid = lax.axis_index("c")
    sid = lax.axis_index("s")
    wid = sid * NC + cid

    _fill_rows(ones_v, C, 1.0)
    _fill_rows(zv, 64, 0.0)

    def zcp(k, _):
        o = pl.multiple_of(sid * ROWS_PS + k * 64, 8)
        pltpu.sync_copy(zv, acc.at[pl.ds(o, 64)])
        return 0
    lax.fori_loop(0, ROWS_PS // 64, zcp, 0)
    plsc.subcore_barrier()

    def body(i, _):
        base = pl.multiple_of(wid * EPW + i * C, 8)
        pltpu.sync_copy(dst_hbm.at[pl.ds(base, C)], dstv)
        pltpu.sync_copy(ones_v, acc.at[dstv], add=True)
        return 0
    lax.fori_loop(0, NITER, body, 0)

    plsc.subcore_barrier()
    off = pl.multiple_of(sid * ROWS_PS, 8)
    pltpu.sync_copy(acc.at[pl.ds(off, ROWS_PS)], out_hbm.at[cid, sid])


@functools.partial(
    pl.kernel,
    out_type=jax.ShapeDtypeStruct((NC, NS, ROWS_PS, D), jnp.float32),
    mesh=_sc_mesh,
    scratch_types=[
        pltpu.VMEM((C,), jnp.int32),
        pltpu.VMEM((C,), jnp.int32),
        pltpu.VMEM((C, D), jnp.float32),
        pltpu.VMEM((64, D), jnp.float32),
        pltpu.VMEM_SHARED((NPAD, D), jnp.float32),
        pltpu.SemaphoreType.DMA,
    ],
)
def _sc_agg(g_hbm, src_hbm, dst_hbm, out_hbm, srcv, dstv, rows, zv, acc, sem):
    cid = lax.axis_index("c")
    sid = lax.axis_index("s")
    wid = sid * NC + cid

    _fill_rows(zv, 64, 0.0)

    def zcp(k, _):
        o = pl.multiple_of(sid * ROWS_PS + k * 64, 8)
        pltpu.sync_copy(zv, acc.at[pl.ds(o, 64)])
        return 0
    lax.fori_loop(0, ROWS_PS // 64, zcp, 0)
    plsc.subcore_barrier()

    def body(i, _):
        base = pl.multiple_of(wid * EPW + i * C, 8)
        pltpu.sync_copy(src_hbm.at[pl.ds(base, C)], srcv)
        pltpu.sync_copy(dst_hbm.at[pl.ds(base, C)], dstv)
        pltpu.async_copy(g_hbm.at[srcv], rows, sem).wait()
        pltpu.sync_copy(rows, acc.at[dstv], add=True)
        return 0
    lax.fori_loop(0, NITER, body, 0)

    plsc.subcore_barrier()
    off = pl.multiple_of(sid * ROWS_PS, 8)
    pltpu.sync_copy(acc.at[pl.ds(off, ROWS_PS)], out_hbm.at[cid, sid])


@functools.partial(
    pl.kernel,
    out_type=[
        jax.ShapeDtypeStruct((NC, BPAD, D), jnp.float32),
        jax.ShapeDtypeStruct((NC, BPAD, D), jnp.float32),
    ],
    mesh=_sc_mesh,
    scratch_types=[
        pltpu.VMEM((PC,), jnp.int32),
        pltpu.VMEM((PC, D), jnp.float32),
        pltpu.VMEM((PC, D), jnp.float32),
        pltpu.VMEM((BPAD, D), jnp.float32),
        pltpu.VMEM_SHARED((BPAD, D), jnp.float32),
        pltpu.VMEM_SHARED((BPAD, D), jnp.float32),
    ],
)
def _sc_pool(h_hbm, batch_hbm, sums_hbm, cnts_hbm,
             bidx, rows, ones_v, zvs, accs, accc):
    cid = lax.axis_index("c")
    sid = lax.axis_index("s")
    wid = sid * NC + cid

    _fill_rows(ones_v, PC, 1.0)
    _fill_rows(zvs, BPAD, 0.0)

    @pl.when(sid == 0)
    def _():
        pltpu.sync_copy(zvs, accs)
        pltpu.sync_copy(zvs, accc)
    plsc.subcore_barrier()

    def body(i, _):
        base = pl.multiple_of(wid * NPPW + i * PC, 8)
        pltpu.sync_copy(batch_hbm.at[pl.ds(base, PC)], bidx)
        pltpu.sync_copy(h_hbm.at[pl.ds(base, PC)], rows)
        pltpu.sync_copy(rows, accs.at[bidx], add=True)
        pltpu.sync_copy(ones_v, accc.at[bidx], add=True)
        return 0
    lax.fori_loop(0, PITER, body, 0)

    plsc.subcore_barrier()

    @pl.when(sid == 0)
    def _():
        pltpu.sync_copy(accs, sums_hbm.at[cid])
        pltpu.sync_copy(accc, cnts_hbm.at[cid])


def _mm_body(x_ref, w_ref, o_ref):
    o_ref[...] = jnp.dot(x_ref[...], w_ref[...],
                         preferred_element_type=jnp.float32)


def _dinv_scale_body(d0_ref, d1_ref, h_ref, g_ref, dinv_ref):
    deg = d0_ref[...][:, 0:1] + d1_ref[...][:, 0:1] + 1.0
    dinv = lax.rsqrt(deg)[:N]
    dinv_ref[...] = dinv
    g_ref[...] = h_ref[...] * dinv


def _layer_body(a0_ref, a1_ref, g_ref, dinv_ref, b_ref, w_ref, o_ref):
    agg = a0_ref[...][:N] + a1_ref[...][:N] + g_ref[...]
    h = jnp.maximum(dinv_ref[...] * agg + b_ref[...], 0.0)
    o_ref[...] = dinv_ref[...] * jnp.dot(h, w_ref[...],
                                         preferred_element_type=jnp.float32)


def _post_body(a0_ref, a1_ref, g_ref, dinv_ref, b_ref, o_ref):
    agg = a0_ref[...][:N] + a1_ref[...][:N] + g_ref[...]
    h = jnp.maximum(dinv_ref[...] * agg + b_ref[...], 0.0)
    o_ref[0:N, :] = h
    o_ref[N:NPAD, :] = jnp.zeros((NPAD - N, D), jnp.float32)


def _final_body(s0_ref, s1_ref, c0_ref, c1_ref, wl_ref, bl_ref, o_ref):
    s = s0_ref[...][:B] + s1_ref[...][:B]
    c = c0_ref[...][:B, 0:1] + c1_ref[...][:B, 0:1]
    pooled = s / jnp.maximum(c, 1.0)
    o_ref[...] = jnp.dot(pooled, wl_ref[...],
                         preferred_element_type=jnp.float32) + bl_ref[...]


def _tc(body, out_shape, *args):
    return pl.pallas_call(body, out_shape=out_shape)(*args)


def kernel(x, edge_index, batch, W1, b1, W2, b2, Wl, bl):
    src = edge_index[0].astype(jnp.int32)
    dst = edge_index[1].astype(jnp.int32)
    batch = batch.astype(jnp.int32)

    degp = _sc_degree(dst).reshape(NC, NPAD, D)
    h1 = _tc(_mm_body, jax.ShapeDtypeStruct((N, D), jnp.float32), x, W1)
    g1, dinv = _tc(
        _dinv_scale_body,
        [jax.ShapeDtypeStruct((N, D), jnp.float32),
         jax.ShapeDtypeStruct((N, 1), jnp.float32)],
        degp[0], degp[1], h1)

    aggp1 = _sc_agg(g1, src, dst).reshape(NC, NPAD, D)
    g2 = _tc(_layer_body, jax.ShapeDtypeStruct((N, D), jnp.float32),
             aggp1[0], aggp1[1], g1, dinv, b1.reshape(1, D), W2)

    aggp2 = _sc_agg(g2, src, dst).reshape(NC, NPAD, D)
    h2o = _tc(_post_body, jax.ShapeDtypeStruct((NPAD, D), jnp.float32),
              aggp2[0], aggp2[1], g2, dinv, b2.reshape(1, D))

    batch_pad = jnp.concatenate(
        [batch, jnp.full((NPAD - N,), B, jnp.int32)])
    sums, cnts = _sc_pool(h2o, batch_pad)

    out = _tc(_final_body, jax.ShapeDtypeStruct((B, 1), jnp.float32),
              sums[0], sums[1], cnts[0], cnts[1], Wl, bl.reshape(1, 1))
    return out


# speedup vs baseline: 20.6683x; 1.7336x over previous
import functools

import jax
import jax.numpy as jnp
from jax import lax
from jax.experimental import pallas as pl
from jax.experimental.pallas import tpu as pltpu
from jax.experimental.pallas import tpu_sc as plsc

N = 10000
E = 320000
D = 128
B = 128

NC = 2
NS = 16
NW = NC * NS
NPAD = 10240
ROWS_PS = NPAD // NS
EPW = E // NW
C = 80
NITER = EPW // C
BPAD = 136
NPPW = NPAD // NW
PC = 64
PITER = NPPW // PC

_sc_mesh = plsc.VectorSubcoreMesh(
    core_axis_name="c", subcore_axis_name="s", num_cores=NC, num_subcores=NS)


def _fill_rows(ref, nrows, value):
    def row(i, _):
        def col(j, _):
            ref[i, pl.ds(j * 16, 16)] = jnp.full((16,), value, jnp.float32)
            return 0
        lax.fori_loop(0, D // 16, col, 0)
        return 0
    lax.fori_loop(0, nrows, row, 0)


@functools.partial(
    pl.kernel,
    out_type=jax.ShapeDtypeStruct((NC, NS, ROWS_PS, D), jnp.float32),
    mesh=_sc_mesh,
    scratch_types=[
        pltpu.VMEM((C,), jnp.int32),
        pltpu.VMEM((C,), jnp.int32),
        pltpu.VMEM((C, D), jnp.float32),
        pltpu.VMEM((64, D), jnp.float32),
        pltpu.VMEM_SHARED((NPAD, D), jnp.float32),
        pltpu.SemaphoreType.DMA,
        pltpu.SemaphoreType.DMA,
        pltpu.SemaphoreType.DMA,
    ],
)
def _sc_degree(dst_hbm, out_hbm, dstv0, dstv1, ones_v, zv, acc,
               sem_i0, sem_i1, sem_s):
    cid = lax.axis_index("c")
    sid = lax.axis_index("s")
    wid = sid * NC + cid
    dstv = (dstv0, dstv1)
    sem_i = (sem_i0, sem_i1)

    _fill_rows(ones_v, C, 1.0)
    _fill_rows(zv, 64, 0.0)

    def zcp(k, _):
        o = pl.multiple_of(sid * ROWS_PS + k * 64, 8)
        pltpu.sync_copy(zv, acc.at[pl.ds(o, 64)])
        return 0
    lax.fori_loop(0, ROWS_PS // 64, zcp, 0)
    plsc.subcore_barrier()

    def idx_base(j):
        return pl.multiple_of(wid * EPW + j * C, 8)

    pltpu.async_copy(dst_hbm.at[pl.ds(idx_base(0), C)], dstv[0], sem_i[0])

    def sub_iter(j, b, first, last):
        ob = 1 - b
        pltpu.make_async_copy(dst_hbm.at[pl.ds(idx_base(j), C)],
                              dstv[b], sem_i[b]).wait()
        if not first:
            pltpu.make_async_copy(ones_v, acc.at[dstv[ob]], sem_s).wait()
        if not last:
            pltpu.async_copy(dst_hbm.at[pl.ds(idx_base(j + 1), C)],
                             dstv[ob], sem_i[ob])
        pltpu.async_copy(ones_v, acc.at[dstv[b]], sem_s, add=True)

    def pair(k, _):
        j0 = k * 2

        @pl.when(k == 0)
        def _():
            sub_iter(j0, 0, True, False)

        @pl.when(k > 0)
        def _():
            sub_iter(j0, 0, False, False)
        sub_iter(j0 + 1, 1, False, False)
        return 0
    lax.fori_loop(0, NITER // 2, pair, 0)
    sub_iter(NITER - 1, 0, False, True)
    pltpu.make_async_copy(ones_v, acc.at[dstv[0]], sem_s).wait()

    plsc.subcore_barrier()
    off = pl.multiple_of(sid * ROWS_PS, 8)
    pltpu.sync_copy(acc.at[pl.ds(off, ROWS_PS)], out_hbm.at[cid, sid])


@functools.partial(
    pl.kernel,
    out_type=jax.ShapeDtypeStruct((NC, NS, ROWS_PS, D), jnp.float32),
    mesh=_sc_mesh,
    scratch_types=[
        pltpu.VMEM((C,), jnp.int32),
        pltpu.VMEM((C,), jnp.int32),
        pltpu.VMEM((C,), jnp.int32),
        pltpu.VMEM((C,), jnp.int32),
        pltpu.VMEM((C, D), jnp.float32),
        pltpu.VMEM((C, D), jnp.float32),
        pltpu.VMEM((64, D), jnp.float32),
        pltpu.VMEM_SHARED((NPAD, D), jnp.float32),
        pltpu.SemaphoreType.DMA,
        pltpu.SemaphoreType.DMA,
        pltpu.SemaphoreType.DMA,
        pltpu.SemaphoreType.DMA,
    ],
)
def _sc_agg(g_hbm, src_hbm, dst_hbm, out_hbm,
            srcv0, srcv1, dstv0, dstv1, rows0, rows1, zv, acc,
            sem_i0, sem_i1, sem_g, sem_s):
    cid = lax.axis_index("c")
    sid = lax.axis_index("s")
    wid = sid * NC + cid
    srcv = (srcv0, srcv1)
    dstv = (dstv0, dstv1)
    rows = (rows0, rows1)
    sem_i = (sem_i0, sem_i1)

    _fill_rows(zv, 64, 0.0)

    def zcp(k, _):
        o = pl.multiple_of(sid * ROWS_PS + k * 64, 8)
        pltpu.sync_copy(zv, acc.at[pl.ds(o, 64)])
        return 0
    lax.fori_loop(0, ROWS_PS // 64, zcp, 0)
    plsc.subcore_barrier()

    def idx_base(j):
        return pl.multiple_of(wid * EPW + j * C, 8)

    def fetch_idx(j, b):
        base = idx_base(j)
        pltpu.async_copy(src_hbm.at[pl.ds(base, C)], srcv[b], sem_i[b])
        pltpu.async_copy(dst_hbm.at[pl.ds(base, C)], dstv[b], sem_i[b])

    fetch_idx(0, 0)

    def sub_iter(j, b, first, last):
        ob = 1 - b
        base = idx_base(j)
        pltpu.make_async_copy(src_hbm.at[pl.ds(base, C)],
                              srcv[b], sem_i[b]).wait()
        pltpu.make_async_copy(dst_hbm.at[pl.ds(base, C)],
                              dstv[b], sem_i[b]).wait()
        gather = pltpu.async_copy(g_hbm.at[srcv[b]], rows[b], sem_g)
        if not first:
            pltpu.make_async_copy(rows[ob], acc.at[dstv[ob]], sem_s).wait()
        if not last:
            fetch_idx(j + 1, ob)
        gather.wait()
        pltpu.async_copy(rows[b], acc.at[dstv[b]], sem_s, add=True)

    def pair(k, _):
        j0 = k * 2

        @pl.when(k == 0)
        def _():
            sub_iter(j0, 0, True, False)

        @pl.when(k > 0)
        def _():
            sub_iter(j0, 0, False, False)
        sub_iter(j0 + 1, 1, False, False)
        return 0
    lax.fori_loop(0, NITER // 2, pair, 0)
    sub_iter(NITER - 1, 0, False, True)
    pltpu.make_async_copy(rows[0], acc.at[dstv[0]], sem_s).wait()

    plsc.subcore_barrier()
    off = pl.multiple_of(sid * ROWS_PS, 8)
    pltpu.sync_copy(acc.at[pl.ds(off, ROWS_PS)], out_hbm.at[cid, sid])


@functools.partial(
    pl.kernel,
    out_type=[
        jax.ShapeDtypeStruct((NC, BPAD, D), jnp.float32),
        jax.ShapeDtypeStruct((NC, BPAD, D), jnp.float32),
    ],
    mesh=_sc_mesh,
    scratch_types=[
        pltpu.VMEM((PC,), jnp.int32),
        pltpu.VMEM((PC, D), jnp.float32),
        pltpu.VMEM((PC, D), jnp.float32),
        pltpu.VMEM((BPAD, D), jnp.float32),
        pltpu.VMEM_SHARED((BPAD, D), jnp.float32),
        pltpu.VMEM_SHARED((BPAD, D), jnp.float32),
    ],
)
def _sc_pool(h_hbm, batch_hbm, sums_hbm, cnts_hbm,
             bidx, rows, ones_v, zvs, accs, accc):
    cid = lax.axis_index("c")
    sid = lax.axis_index("s")
    wid = sid * NC + cid

    _fill_rows(ones_v, PC, 1.0)
    _fill_rows(zvs, BPAD, 0.0)

    @pl.when(sid == 0)
    def _():
        pltpu.sync_copy(zvs, accs)
        pltpu.sync_copy(zvs, accc)
    plsc.subcore_barrier()

    def body(i, _):
        base = pl.multiple_of(wid * NPPW + i * PC, 8)
        pltpu.sync_copy(batch_hbm.at[pl.ds(base, PC)], bidx)
        pltpu.sync_copy(h_hbm.at[pl.ds(base, PC)], rows)
        pltpu.sync_copy(rows, accs.at[bidx], add=True)
        pltpu.sync_copy(ones_v, accc.at[bidx], add=True)
        return 0
    lax.fori_loop(0, PITER, body, 0)

    plsc.subcore_barrier()

    @pl.when(sid == 0)
    def _():
        pltpu.sync_copy(accs, sums_hbm.at[cid])
        pltpu.sync_copy(accc, cnts_hbm.at[cid])


def _mm_dinv_body(x_ref, w_ref, d0_ref, d1_ref, g_ref, dinv_ref):
    h = jnp.dot(x_ref[...], w_ref[...], preferred_element_type=jnp.float32)
    deg = d0_ref[...][:, 0:1] + d1_ref[...][:, 0:1] + 1.0
    dinv = lax.rsqrt(deg)[:N]
    dinv_ref[...] = dinv
    g_ref[...] = h * dinv


def _layer_body(a0_ref, a1_ref, g_ref, dinv_ref, b_ref, w_ref, o_ref):
    agg = a0_ref[...][:N] + a1_ref[...][:N] + g_ref[...]
    h = jnp.maximum(dinv_ref[...] * agg + b_ref[...], 0.0)
    o_ref[...] = dinv_ref[...] * jnp.dot(h, w_ref[...],
                                         preferred_element_type=jnp.float32)


def _post_body(a0_ref, a1_ref, g_ref, dinv_ref, b_ref, o_ref):
    agg = a0_ref[...][:N] + a1_ref[...][:N] + g_ref[...]
    h = jnp.maximum(dinv_ref[...] * agg + b_ref[...], 0.0)
    o_ref[0:N, :] = h
    o_ref[N:NPAD, :] = jnp.zeros((NPAD - N, D), jnp.float32)


def _final_body(s0_ref, s1_ref, c0_ref, c1_ref, wl_ref, bl_ref, o_ref):
    s = s0_ref[...][:B] + s1_ref[...][:B]
    c = c0_ref[...][:B, 0:1] + c1_ref[...][:B, 0:1]
    pooled = s / jnp.maximum(c, 1.0)
    o_ref[...] = jnp.dot(pooled, wl_ref[...],
                         preferred_element_type=jnp.float32) + bl_ref[...]


def _tc(body, out_shape, *args):
    return pl.pallas_call(body, out_shape=out_shape)(*args)


def kernel(x, edge_index, batch, W1, b1, W2, b2, Wl, bl):
    src = edge_index[0].astype(jnp.int32)
    dst = edge_index[1].astype(jnp.int32)
    batch = batch.astype(jnp.int32)

    degp = _sc_degree(dst).reshape(NC, NPAD, D)
    g1, dinv = _tc(
        _mm_dinv_body,
        [jax.ShapeDtypeStruct((N, D), jnp.float32),
         jax.ShapeDtypeStruct((N, 1), jnp.float32)],
        x, W1, degp[0], degp[1])

    aggp1 = _sc_agg(g1, src, dst).reshape(NC, NPAD, D)
    g2 = _tc(_layer_body, jax.ShapeDtypeStruct((N, D), jnp.float32),
             aggp1[0], aggp1[1], g1, dinv, b1.reshape(1, D), W2)

    aggp2 = _sc_agg(g2, src, dst).reshape(NC, NPAD, D)
    h2o = _tc(_post_body, jax.ShapeDtypeStruct((NPAD, D), jnp.float32),
              aggp2[0], aggp2[1], g2, dinv, b2.reshape(1, D))

    batch_pad = jnp.concatenate(
        [batch, jnp.full((NPAD - N,), B, jnp.int32)])
    sums, cnts = _sc_pool(h2o, batch_pad)

    out = _tc(_final_body, jax.ShapeDtypeStruct((B, 1), jnp.float32),
              sums[0], sums[1], cnts[0], cnts[1], Wl, bl.reshape(1, 1))
    return out


# speedup vs baseline: 20.7449x; 1.0037x over previous
import functools

import jax
import jax.numpy as jnp
from jax import lax
from jax.experimental import pallas as pl
from jax.experimental.pallas import tpu as pltpu
from jax.experimental.pallas import tpu_sc as plsc

N = 10000
E = 320000
D = 128
B = 128

NC = 2
NS = 16
NW = NC * NS
NPAD = 10240
ROWS_PS = NPAD // NS
EPW = E // NW
C = 80
NITER = EPW // C
BPAD = 136
NPPW = NPAD // NW
PC = 64
PITER = NPPW // PC

_sc_mesh = plsc.VectorSubcoreMesh(
    core_axis_name="c", subcore_axis_name="s", num_cores=NC, num_subcores=NS)


def _fill_rows(ref, nrows, value):
    def row(i, _):
        def col(j, _):
            ref[i, pl.ds(j * 16, 16)] = jnp.full((16,), value, jnp.float32)
            return 0
        lax.fori_loop(0, D // 16, col, 0)
        return 0
    lax.fori_loop(0, nrows, row, 0)


NB = 4


@functools.partial(
    pl.kernel,
    out_type=jax.ShapeDtypeStruct((NC, NS, ROWS_PS, D), jnp.float32),
    mesh=_sc_mesh,
    scratch_types=(
        [pltpu.VMEM((C,), jnp.int32) for _ in range(NB)]
        + [pltpu.VMEM((C, D), jnp.float32),
           pltpu.VMEM((64, D), jnp.float32),
           pltpu.VMEM_SHARED((NPAD, D), jnp.float32)]
        + [pltpu.SemaphoreType.DMA for _ in range(2 * NB)]
    ),
)
def _sc_degree(dst_hbm, out_hbm, *refs):
    dstv = refs[0:NB]
    ones_v, zv, acc = refs[NB:NB + 3]
    sem_i = refs[NB + 3:NB + 3 + NB]
    sem_s = refs[NB + 3 + NB:NB + 3 + 2 * NB]
    cid = lax.axis_index("c")
    sid = lax.axis_index("s")
    wid = sid * NC + cid

    _fill_rows(ones_v, C, 1.0)
    _fill_rows(zv, 64, 0.0)

    def zcp(k, _):
        o = pl.multiple_of(sid * ROWS_PS + k * 64, 8)
        pltpu.sync_copy(zv, acc.at[pl.ds(o, 64)])
        return 0
    lax.fori_loop(0, ROWS_PS // 64, zcp, 0)
    plsc.subcore_barrier()

    def idx_base(j):
        return pl.multiple_of(wid * EPW + j * C, 8)

    def fetch_idx(j, b):
        pltpu.async_copy(dst_hbm.at[pl.ds(idx_base(j), C)], dstv[b], sem_i[b])

    fetch_idx(0, 0)

    def sub_iter(j, b, skip_swait, last):
        nb = (b + 1) % NB
        pltpu.make_async_copy(dst_hbm.at[pl.ds(idx_base(j), C)],
                              dstv[b], sem_i[b]).wait()
        if not skip_swait:
            pltpu.make_async_copy(ones_v, acc.at[dstv[nb]], sem_s[nb]).wait()
        if not last:
            fetch_idx(j + 1, nb)
        pltpu.async_copy(ones_v, acc.at[dstv[b]], sem_s[b], add=True)

    def quad(k, _):
        j0 = k * NB

        @pl.when(k == 0)
        def _():
            for u in range(NB):
                sub_iter(j0 + u, u, u < NB - 1, False)

        @pl.when(k > 0)
        def _():
            for u in range(NB):
                sub_iter(j0 + u, u, False, False)
        return 0
    lax.fori_loop(0, NITER // NB, quad, 0)
    sub_iter(NITER - 1, (NITER - 1) % NB, False, True)
    for j in (NITER - 3, NITER - 2, NITER - 1):
        b = j % NB
        pltpu.make_async_copy(ones_v, acc.at[dstv[b]], sem_s[b]).wait()

    plsc.subcore_barrier()
    off = pl.multiple_of(sid * ROWS_PS, 8)
    pltpu.sync_copy(acc.at[pl.ds(off, ROWS_PS)], out_hbm.at[cid, sid])


@functools.partial(
    pl.kernel,
    out_type=jax.ShapeDtypeStruct((NC, NS, ROWS_PS, D), jnp.float32),
    mesh=_sc_mesh,
    scratch_types=(
        [pltpu.VMEM((C,), jnp.int32) for _ in range(2 * NB)]
        + [pltpu.VMEM((C, D), jnp.float32) for _ in range(NB)]
        + [pltpu.VMEM_SHARED((NPAD, D), jnp.float32)]
        + [pltpu.SemaphoreType.DMA for _ in range(2 * NB + 1)]
    ),
)
def _sc_agg(g_hbm, src_hbm, dst_hbm, out_hbm, *refs):
    srcv = refs[0:NB]
    dstv = refs[NB:2 * NB]
    rows = refs[2 * NB:3 * NB]
    acc = refs[3 * NB]
    sem_i = refs[3 * NB + 1:3 * NB + 1 + NB]
    sem_s = refs[3 * NB + 1 + NB:3 * NB + 1 + 2 * NB]
    sem_g = refs[3 * NB + 1 + 2 * NB]
    cid = lax.axis_index("c")
    sid = lax.axis_index("s")
    wid = sid * NC + cid

    _fill_rows(rows[0], C, 0.0)

    def zcp(k, _):
        o = pl.multiple_of(sid * ROWS_PS + k * C, 8)
        pltpu.sync_copy(rows[0], acc.at[pl.ds(o, C)])
        return 0
    lax.fori_loop(0, ROWS_PS // C, zcp, 0)
    plsc.subcore_barrier()

    def idx_base(j):
        return pl.multiple_of(wid * EPW + j * C, 8)

    def fetch_idx(j, b):
        base = idx_base(j)
        pltpu.async_copy(src_hbm.at[pl.ds(base, C)], srcv[b], sem_i[b])
        pltpu.async_copy(dst_hbm.at[pl.ds(base, C)], dstv[b], sem_i[b])

    fetch_idx(0, 0)

    def sub_iter(j, b, skip_swait, last):
        nb = (b + 1) % NB
        base = idx_base(j)
        pltpu.make_async_copy(src_hbm.at[pl.ds(base, C)],
                              srcv[b], sem_i[b]).wait()
        pltpu.make_async_copy(dst_hbm.at[pl.ds(base, C)],
                              dstv[b], sem_i[b]).wait()
        gather = pltpu.async_copy(g_hbm.at[srcv[b]], rows[b], sem_g)
        if not skip_swait:
            pltpu.make_async_copy(rows[nb], acc.at[dstv[nb]], sem_s[nb]).wait()
        if not last:
            fetch_idx(j + 1, nb)
        gather.wait()
        pltpu.async_copy(rows[b], acc.at[dstv[b]], sem_s[b], add=True)

    def quad(k, _):
        j0 = k * NB

        @pl.when(k == 0)
        def _():
            for u in range(NB):
                sub_iter(j0 + u, u, u < NB - 1, False)

        @pl.when(k > 0)
        def _():
            for u in range(NB):
                sub_iter(j0 + u, u, False, False)
        return 0
    lax.fori_loop(0, NITER // NB, quad, 0)
    sub_iter(NITER - 1, (NITER - 1) % NB, False, True)
    for j in (NITER - 3, NITER - 2, NITER - 1):
        b = j % NB
        pltpu.make_async_copy(rows[b], acc.at[dstv[b]], sem_s[b]).wait()

    plsc.subcore_barrier()
    off = pl.multiple_of(sid * ROWS_PS, 8)
    pltpu.sync_copy(acc.at[pl.ds(off, ROWS_PS)], out_hbm.at[cid, sid])


@functools.partial(
    pl.kernel,
    out_type=[
        jax.ShapeDtypeStruct((NC, BPAD, D), jnp.float32),
        jax.ShapeDtypeStruct((NC, BPAD, D), jnp.float32),
    ],
    mesh=_sc_mesh,
    scratch_types=[
        pltpu.VMEM((PC,), jnp.int32),
        pltpu.VMEM((PC, D), jnp.float32),
        pltpu.VMEM((PC, D), jnp.float32),
        pltpu.VMEM((BPAD, D), jnp.float32),
        pltpu.VMEM_SHARED((BPAD, D), jnp.float32),
        pltpu.VMEM_SHARED((BPAD, D), jnp.float32),
    ],
)
def _sc_pool(h_hbm, batch_hbm, sums_hbm, cnts_hbm,
             bidx, rows, ones_v, zvs, accs, accc):
    cid = lax.axis_index("c")
    sid = lax.axis_index("s")
    wid = sid * NC + cid

    _fill_rows(ones_v, PC, 1.0)
    _fill_rows(zvs, BPAD, 0.0)

    @pl.when(sid == 0)
    def _():
        pltpu.sync_copy(zvs, accs)
        pltpu.sync_copy(zvs, accc)
    plsc.subcore_barrier()

    def body(i, _):
        base = pl.multiple_of(wid * NPPW + i * PC, 8)
        pltpu.sync_copy(batch_hbm.at[pl.ds(base, PC)], bidx)
        pltpu.sync_copy(h_hbm.at[pl.ds(base, PC)], rows)
        pltpu.sync_copy(rows, accs.at[bidx], add=True)
        pltpu.sync_copy(ones_v, accc.at[bidx], add=True)
        return 0
    lax.fori_loop(0, PITER, body, 0)

    plsc.subcore_barrier()

    @pl.when(sid == 0)
    def _():
        pltpu.sync_copy(accs, sums_hbm.at[cid])
        pltpu.sync_copy(accc, cnts_hbm.at[cid])


def _mm_dinv_body(x_ref, w_ref, d0_ref, d1_ref, g_ref, dinv_ref):
    h = jnp.dot(x_ref[...], w_ref[...], preferred_element_type=jnp.float32)
    deg = d0_ref[...][:, 0:1] + d1_ref[...][:, 0:1] + 1.0
    dinv = lax.rsqrt(deg)[:N]
    dinv_ref[...] = dinv
    g_ref[...] = h * dinv


def _layer_body(a0_ref, a1_ref, g_ref, dinv_ref, b_ref, w_ref, o_ref):
    agg = a0_ref[...][:N] + a1_ref[...][:N] + g_ref[...]
    h = jnp.maximum(dinv_ref[...] * agg + b_ref[...], 0.0)
    o_ref[...] = dinv_ref[...] * jnp.dot(h, w_ref[...],
                                         preferred_element_type=jnp.float32)


def _post_body(a0_ref, a1_ref, g_ref, dinv_ref, b_ref, o_ref):
    agg = a0_ref[...][:N] + a1_ref[...][:N] + g_ref[...]
    h = jnp.maximum(dinv_ref[...] * agg + b_ref[...], 0.0)
    o_ref[0:N, :] = h
    o_ref[N:NPAD, :] = jnp.zeros((NPAD - N, D), jnp.float32)


def _final_body(s0_ref, s1_ref, c0_ref, c1_ref, wl_ref, bl_ref, o_ref):
    s = s0_ref[...][:B] + s1_ref[...][:B]
    c = c0_ref[...][:B, 0:1] + c1_ref[...][:B, 0:1]
    pooled = s / jnp.maximum(c, 1.0)
    o_ref[...] = jnp.dot(pooled, wl_ref[...],
                         preferred_element_type=jnp.float32) + bl_ref[...]


def _tc(body, out_shape, *args):
    return pl.pallas_call(body, out_shape=out_shape)(*args)


def kernel(x, edge_index, batch, W1, b1, W2, b2, Wl, bl):
    src = edge_index[0].astype(jnp.int32)
    dst = edge_index[1].astype(jnp.int32)
    batch = batch.astype(jnp.int32)

    degp = _sc_degree(dst).reshape(NC, NPAD, D)
    g1, dinv = _tc(
        _mm_dinv_body,
        [jax.ShapeDtypeStruct((N, D), jnp.float32),
         jax.ShapeDtypeStruct((N, 1), jnp.float32)],
        x, W1, degp[0], degp[1])

    aggp1 = _sc_agg(g1, src, dst).reshape(NC, NPAD, D)
    g2 = _tc(_layer_body, jax.ShapeDtypeStruct((N, D), jnp.float32),
             aggp1[0], aggp1[1], g1, dinv, b1.reshape(1, D), W2)

    aggp2 = _sc_agg(g2, src, dst).reshape(NC, NPAD, D)
    h2o = _tc(_post_body, jax.ShapeDtypeStruct((NPAD, D), jnp.float32),
              aggp2[0], aggp2[1], g2, dinv, b2.reshape(1, D))

    batch_pad = jnp.concatenate(
        [batch, jnp.full((NPAD - N,), B, jnp.int32)])
    sums, cnts = _sc_pool(h2o, batch_pad)

    out = _tc(_final_body, jax.ShapeDtypeStruct((B, 1), jnp.float32),
              sums[0], sums[1], cnts[0], cnts[1], Wl, bl.reshape(1, 1))
    return out
